# baseline (speedup 1.0000x reference)
import functools

import jax
import jax.numpy as jnp
import numpy as np
from jax.experimental import pallas as pl
from jax.experimental.pallas import tpu as pltpu

B, T, N = 128, 64, 9
FEAT = 16
L, H, C = 3, 4, 32
HID = H * C
ENC = 128
ROWS_PER_B = T * N

_node_types = np.array([0, 0, 0, 0, 0, 0, 0, 1, 2])
_onehot = np.eye(3, dtype=np.float32)[_node_types]
_dist = np.abs(np.arange(N) - 7).astype(np.float32)
_STATIC = np.concatenate([_onehot, _dist[:, None]], axis=1)

_NEG = -1e30


def _gat_kernel(j0, j1, j2, j3, j4, j5, j6, j7, j8,
                enc_w, enc_b, Wl, bl, Wr, br, att, We, Wres, bo, ln_g, ln_b,
                static, out_ref, *, bb):
    bt = bb * T
    R = N * bt
    feats = [j0, j1, j2, j3, j4, j5, j6, j7, j8]

    blk = (jax.lax.broadcasted_iota(jnp.int32, (HID, HID), 0) // C ==
           jax.lax.broadcasted_iota(jnp.int32, (HID, HID), 1) // C
           ).astype(jnp.float32)
    ones = jnp.ones((HID, HID), jnp.float32)

    xs = []
    ps = []
    for i in range(N):
        f = feats[i][...].reshape(bt, FEAT)
        const = static[i:i + 1, :] @ enc_w[i, FEAT:, :] + enc_b[i][None, :]
        xs.append(jnp.dot(f, enc_w[i, :FEAT, :],
                          preferred_element_type=jnp.float32) + const)
        ps.append(f[:, :3])
    x = jnp.concatenate(xs, axis=0)
    pos = jnp.concatenate(ps, axis=0)

    def sh_l(a, w):
        return jnp.concatenate([jnp.zeros((bt, w), jnp.float32), a[:-bt]], 0)

    def sh_r(a, w):
        return jnp.concatenate([a[bt:], jnp.zeros((bt, w), jnp.float32)], 0)

    def sh_t(a, w):
        return jnp.concatenate([jnp.zeros((1, w), jnp.float32), a[:-1]], 0)

    r0 = jax.lax.broadcasted_iota(jnp.int32, (R, HID), 0)
    am_l = jnp.where(r0 >= bt, 0.0, _NEG)
    am_r = jnp.where(r0 < R - bt, 0.0, _NEG)
    am_t = jnp.where(r0 % T != 0, 0.0, _NEG)

    r1 = jax.lax.broadcasted_iota(jnp.int32, (R, 1), 0)
    m_l3 = (r1 >= bt).astype(jnp.float32)
    m_r3 = (r1 < R - bt).astype(jnp.float32)
    m_t3 = (r1 % T != 0).astype(jnp.float32)
    deg = m_l3 + m_r3 + m_t3

    pos_l = sh_l(pos, 3)
    pos_r = sh_r(pos, 3)
    pos_t = sh_t(pos, 3)
    dp_smean = (m_l3 * pos_l + m_r3 * pos_r + m_t3 * pos_t) / deg

    eye = (jax.lax.broadcasted_iota(jnp.int32, (HID, HID), 0) ==
           jax.lax.broadcasted_iota(jnp.int32, (HID, HID), 1)
           ).astype(jnp.float32)

    for l in range(L):
        wcat = jnp.concatenate([Wl[l], Wr[l], Wres[l]], axis=1)
        bcat = jnp.concatenate([bl[l], br[l], bo[l]], axis=0)[None, :]
        xw = jnp.dot(x, wcat, preferred_element_type=jnp.float32) + bcat
        xl = xw[:, :HID]
        xr = xw[:, HID:2 * HID]
        res = xw[:, 2 * HID:]

        g = jnp.dot(pos, We[l], preferred_element_type=jnp.float32)
        g_l = sh_l(g, HID)
        g_r = sh_r(g, HID)
        g_t = sh_t(g, HID)
        eesm = jnp.dot(dp_smean, We[l], preferred_element_type=jnp.float32)

        P = xr + g
        xl_l = sh_l(xl, HID)
        xl_r = sh_r(xl, HID)
        xl_t = sh_t(xl, HID)

        attf = att[l].reshape(1, HID) * np.float32(1.4426950408889634)
        attblk = jnp.dot(eye * attf, blk, preferred_element_type=jnp.float32)

        def logit(m):
            m = jnp.maximum(m, 0.2 * m)
            return jnp.dot(m, attblk, preferred_element_type=jnp.float32)

        e_l = jnp.exp2(logit(P + xl_l - g_l) + am_l)
        e_r = jnp.exp2(logit(P + xl_r - g_r) + am_r)
        e_t = jnp.exp2(logit(P + xl_t - g_t) + am_t)
        e_s = jnp.exp2(logit(P + xl - eesm))
        inv = 1.0 / (e_l + e_r + e_t + e_s + 1e-16)

        out = (e_l * xl_l + e_r * xl_r + e_t * xl_t +
               e_s * xl) * inv + res

        s1 = jnp.dot(out, ones, preferred_element_type=jnp.float32)
        s2 = jnp.dot(out * out, ones, preferred_element_type=jnp.float32)
        mu = s1 * (1.0 / HID)
        var = s2 * (1.0 / HID) - mu * mu
        y = (out - mu) * jax.lax.rsqrt(var + 1e-5) * ln_g[l][None, :] + ln_b[l][None, :]
        x = y * jax.nn.sigmoid(y)

    pt = x.reshape(N * bb, T, HID).sum(axis=1)
    pooled = pt.reshape(N, bb, HID).sum(axis=0) * (1.0 / ROWS_PER_B)
    out_ref[...] = pooled


@jax.jit
def kernel(joint_0, joint_1, joint_2, joint_3, joint_4, joint_5, joint_6,
           eef, object, enc_w, enc_b, Wl, bl, Wr, br, att, We, Wres, bo,
           ln_g, ln_b):
    bb = 16
    k = B // bb
    feats = [joint_0, joint_1, joint_2, joint_3, joint_4, joint_5, joint_6,
             eef, object]
    feat_spec = pl.BlockSpec((bb, T, FEAT), lambda g: (g, 0, 0))

    def full(shape):
        nd = len(shape)
        return pl.BlockSpec(shape, lambda g: (0,) * nd)

    in_specs = [feat_spec] * N + [
        full(enc_w.shape), full(enc_b.shape), full(Wl.shape), full(bl.shape),
        full(Wr.shape), full(br.shape), full(att.shape), full(We.shape),
        full(Wres.shape), full(bo.shape), full(ln_g.shape), full(ln_b.shape),
        full((N, 4)),
    ]
    static = jnp.asarray(_STATIC)
    out = pl.pallas_call(
        functools.partial(_gat_kernel, bb=bb),
        grid=(k,),
        compiler_params=pltpu.CompilerParams(
            dimension_semantics=("parallel",)),
        in_specs=in_specs,
        out_specs=pl.BlockSpec((bb, HID), lambda g: (g, 0)),
        out_shape=jax.ShapeDtypeStruct((B, HID), jnp.float32),
    )(*feats, enc_w, enc_b, Wl, bl, Wr, br, att, We, Wres, bo, ln_g, ln_b,
      static)
    return out


# speedup vs baseline: 1.2735x; 1.2735x over previous
import functools

import jax
import jax.numpy as jnp
import numpy as np
from jax.experimental import pallas as pl
from jax.experimental.pallas import tpu as pltpu

B, T, N = 128, 64, 9
FEAT = 16
L, H, C = 3, 4, 32
HID = H * C
ENC = 128
ROWS_PER_B = T * N

_node_types = np.array([0, 0, 0, 0, 0, 0, 0, 1, 2])
_onehot = np.eye(3, dtype=np.float32)[_node_types]
_dist = np.abs(np.arange(N) - 7).astype(np.float32)
_STATIC = np.concatenate([_onehot, _dist[:, None]], axis=1)

_NEG = -1e30


def _gat_kernel(j0, j1, j2, j3, j4, j5, j6, j7, j8,
                enc_w, enc_b, Wl, bl, Wr, br, att, We, Wres, bo, ln_g, ln_b,
                static, out_ref, *, bb):
    bt = bb * T
    R = N * bt
    feats = [j0, j1, j2, j3, j4, j5, j6, j7, j8]

    blk = (jax.lax.broadcasted_iota(jnp.int32, (HID, HID), 0) // C ==
           jax.lax.broadcasted_iota(jnp.int32, (HID, HID), 1) // C
           ).astype(jnp.float32)
    ones = jnp.ones((HID, HID), jnp.float32)

    xs = []
    ps = []
    for i in range(N):
        f = feats[i][...].reshape(bt, FEAT)
        const = static[i:i + 1, :] @ enc_w[i, FEAT:, :] + enc_b[i][None, :]
        xs.append(jnp.dot(f, enc_w[i, :FEAT, :],
                          preferred_element_type=jnp.float32) + const)
        ps.append(f[:, :3])
    x = jnp.concatenate(xs, axis=0)
    pos = jnp.concatenate(ps, axis=0)

    def sh_l(a, w):
        return jnp.concatenate([jnp.zeros((bt, w), jnp.float32), a[:-bt]], 0)

    def sh_r(a, w):
        return jnp.concatenate([a[bt:], jnp.zeros((bt, w), jnp.float32)], 0)

    def sh_t(a, w):
        return jnp.concatenate([jnp.zeros((1, w), jnp.float32), a[:-1]], 0)

    r0 = jax.lax.broadcasted_iota(jnp.int32, (R, HID), 0)
    am_l = jnp.where(r0 >= bt, 0.0, _NEG)
    am_r = jnp.where(r0 < R - bt, 0.0, _NEG)
    am_t = jnp.where(r0 % T != 0, 0.0, _NEG)

    r1 = jax.lax.broadcasted_iota(jnp.int32, (R, 1), 0)
    m_l3 = (r1 >= bt).astype(jnp.float32)
    m_r3 = (r1 < R - bt).astype(jnp.float32)
    m_t3 = (r1 % T != 0).astype(jnp.float32)
    deg = m_l3 + m_r3 + m_t3

    pos_l = sh_l(pos, 3)
    pos_r = sh_r(pos, 3)
    pos_t = sh_t(pos, 3)
    dp_smean = (m_l3 * pos_l + m_r3 * pos_r + m_t3 * pos_t) / deg

    eye = (jax.lax.broadcasted_iota(jnp.int32, (HID, HID), 0) ==
           jax.lax.broadcasted_iota(jnp.int32, (HID, HID), 1)
           ).astype(jnp.float32)

    for l in range(L):
        wcat = jnp.concatenate([Wl[l], Wr[l], Wres[l]], axis=1)
        bcat = jnp.concatenate([bl[l], br[l], bo[l]], axis=0)[None, :]
        xw = jnp.dot(x, wcat, preferred_element_type=jnp.float32) + bcat
        xl = xw[:, :HID]
        xr = xw[:, HID:2 * HID]
        res = xw[:, 2 * HID:]

        g = jnp.dot(pos, We[l], preferred_element_type=jnp.float32)
        g_l = sh_l(g, HID)
        g_r = sh_r(g, HID)
        g_t = sh_t(g, HID)
        eesm = jnp.dot(dp_smean, We[l], preferred_element_type=jnp.float32)

        P = xr + g
        xl_l = sh_l(xl, HID)
        xl_r = sh_r(xl, HID)
        xl_t = sh_t(xl, HID)

        attf = att[l].reshape(1, HID) * np.float32(1.4426950408889634)
        attblk = jnp.dot(eye * attf, blk, preferred_element_type=jnp.float32)

        def logit(m):
            m = jnp.maximum(m, 0.2 * m)
            return jnp.dot(m, attblk, preferred_element_type=jnp.float32)

        e_l = jnp.exp2(logit(P + xl_l - g_l) + am_l)
        e_r = jnp.exp2(logit(P + xl_r - g_r) + am_r)
        e_t = jnp.exp2(logit(P + xl_t - g_t) + am_t)
        e_s = jnp.exp2(logit(P + xl - eesm))
        inv = 1.0 / (e_l + e_r + e_t + e_s + 1e-16)

        out = (e_l * xl_l + e_r * xl_r + e_t * xl_t +
               e_s * xl) * inv + res

        s1 = jnp.dot(out, ones, preferred_element_type=jnp.float32)
        s2 = jnp.dot(out * out, ones, preferred_element_type=jnp.float32)
        mu = s1 * (1.0 / HID)
        var = s2 * (1.0 / HID) - mu * mu
        y = (out - mu) * jax.lax.rsqrt(var + 1e-5) * ln_g[l][None, :] + ln_b[l][None, :]
        x = y * jax.nn.sigmoid(y)

    pt = x.reshape(N * bb, T, HID).sum(axis=1)
    pooled = pt.reshape(N, bb, HID).sum(axis=0) * (1.0 / ROWS_PER_B)
    out_ref[...] = pooled


@jax.jit
def kernel(joint_0, joint_1, joint_2, joint_3, joint_4, joint_5, joint_6,
           eef, object, enc_w, enc_b, Wl, bl, Wr, br, att, We, Wres, bo,
           ln_g, ln_b):
    bb = 8
    k = B // bb
    feats = [joint_0, joint_1, joint_2, joint_3, joint_4, joint_5, joint_6,
             eef, object]
    feat_spec = pl.BlockSpec((bb, T, FEAT), lambda g: (g, 0, 0))

    def full(shape):
        nd = len(shape)
        return pl.BlockSpec(shape, lambda g: (0,) * nd)

    in_specs = [feat_spec] * N + [
        full(enc_w.shape), full(enc_b.shape), full(Wl.shape), full(bl.shape),
        full(Wr.shape), full(br.shape), full(att.shape), full(We.shape),
        full(Wres.shape), full(bo.shape), full(ln_g.shape), full(ln_b.shape),
        full((N, 4)),
    ]
    static = jnp.asarray(_STATIC)
    out = pl.pallas_call(
        functools.partial(_gat_kernel, bb=bb),
        grid=(k,),
        compiler_params=pltpu.CompilerParams(
            dimension_semantics=("parallel",)),
        in_specs=in_specs,
        out_specs=pl.BlockSpec((bb, HID), lambda g: (g, 0)),
        out_shape=jax.ShapeDtypeStruct((B, HID), jnp.float32),
    )(*feats, enc_w, enc_b, Wl, bl, Wr, br, att, We, Wres, bo, ln_g, ln_b,
      static)
    return out


# speedup vs baseline: 1.2736x; 1.0001x over previous
import functools

import jax
import jax.numpy as jnp
import numpy as np
from jax.experimental import pallas as pl
from jax.experimental.pallas import tpu as pltpu

B, T, N = 128, 64, 9
FEAT = 16
L, H, C = 3, 4, 32
HID = H * C
ENC = 128
ROWS_PER_B = T * N

_node_types = np.array([0, 0, 0, 0, 0, 0, 0, 1, 2])
_onehot = np.eye(3, dtype=np.float32)[_node_types]
_dist = np.abs(np.arange(N) - 7).astype(np.float32)
_STATIC = np.concatenate([_onehot, _dist[:, None]], axis=1)

_NEG = -1e30


def _gat_kernel(j0, j1, j2, j3, j4, j5, j6, j7, j8,
                enc_w, enc_b, Wl, bl, Wr, br, att, We, Wres, bo, ln_g, ln_b,
                static, out_ref, *, bb):
    bt = bb * T
    R = N * bt
    feats = [j0, j1, j2, j3, j4, j5, j6, j7, j8]

    blk = (jax.lax.broadcasted_iota(jnp.int32, (HID, HID), 0) // C ==
           jax.lax.broadcasted_iota(jnp.int32, (HID, HID), 1) // C
           ).astype(jnp.float32)
    ones = jnp.ones((HID, HID), jnp.float32)

    xs = []
    ps = []
    for i in range(N):
        f = feats[i][...].reshape(bt, FEAT)
        const = static[i:i + 1, :] @ enc_w[i, FEAT:, :] + enc_b[i][None, :]
        xs.append(jnp.dot(f, enc_w[i, :FEAT, :],
                          preferred_element_type=jnp.float32) + const)
        ps.append(f[:, :3])
    x = jnp.concatenate(xs, axis=0)
    pos = jnp.concatenate(ps, axis=0)

    def sh_l(a, w):
        return jnp.concatenate([jnp.zeros((bt, w), jnp.float32), a[:-bt]], 0)

    def sh_r(a, w):
        return jnp.concatenate([a[bt:], jnp.zeros((bt, w), jnp.float32)], 0)

    def sh_t(a, w):
        return jnp.concatenate([jnp.zeros((1, w), jnp.float32), a[:-1]], 0)

    r0 = jax.lax.broadcasted_iota(jnp.int32, (R, HID), 0)
    am_l = jnp.where(r0 >= bt, 0.0, _NEG)
    am_r = jnp.where(r0 < R - bt, 0.0, _NEG)
    am_t = jnp.where(r0 % T != 0, 0.0, _NEG)

    r1 = jax.lax.broadcasted_iota(jnp.int32, (R, 1), 0)
    m_l3 = (r1 >= bt).astype(jnp.float32)
    m_r3 = (r1 < R - bt).astype(jnp.float32)
    m_t3 = (r1 % T != 0).astype(jnp.float32)
    deg = m_l3 + m_r3 + m_t3

    pos_l = sh_l(pos, 3)
    pos_r = sh_r(pos, 3)
    pos_t = sh_t(pos, 3)
    dp_smean = (m_l3 * pos_l + m_r3 * pos_r + m_t3 * pos_t) / deg

    eye = (jax.lax.broadcasted_iota(jnp.int32, (HID, HID), 0) ==
           jax.lax.broadcasted_iota(jnp.int32, (HID, HID), 1)
           ).astype(jnp.float32)

    for l in range(L):
        wcat = jnp.concatenate([Wl[l], Wr[l], Wres[l]], axis=1)
        bcat = jnp.concatenate([bl[l], br[l], bo[l]], axis=0)[None, :]
        xw = jnp.dot(x, wcat, preferred_element_type=jnp.float32) + bcat
        xl = xw[:, :HID]
        xr = xw[:, HID:2 * HID]
        res = xw[:, 2 * HID:]

        g = jnp.dot(pos, We[l], preferred_element_type=jnp.float32)
        eesm = jnp.dot(dp_smean, We[l], preferred_element_type=jnp.float32)

        zpad = jnp.zeros((bt, HID), jnp.float32)
        xlz = jnp.concatenate([zpad, xl, zpad], 0)
        gz = jnp.concatenate([zpad, g, zpad], 0)
        xl_l = xlz[:R]
        xl_r = xlz[2 * bt:]
        xl_t = xlz[bt - 1:bt - 1 + R]
        g_l = gz[:R]
        g_r = gz[2 * bt:]
        g_t = gz[bt - 1:bt - 1 + R]

        P = xr + g

        attf = att[l].reshape(1, HID) * np.float32(1.4426950408889634)
        attblk = jnp.dot(eye * attf, blk, preferred_element_type=jnp.float32)

        def logit(m):
            m = jnp.maximum(m, 0.2 * m)
            return jnp.dot(m, attblk, preferred_element_type=jnp.float32)

        e_l = jnp.exp2(logit(P + xl_l - g_l) + am_l)
        e_r = jnp.exp2(logit(P + xl_r - g_r) + am_r)
        e_t = jnp.exp2(logit(P + xl_t - g_t) + am_t)
        e_s = jnp.exp2(logit(P + xl - eesm))
        inv = 1.0 / (e_l + e_r + e_t + e_s + 1e-16)

        out = (e_l * xl_l + e_r * xl_r + e_t * xl_t +
               e_s * xl) * inv + res

        s1 = jnp.dot(out, ones, preferred_element_type=jnp.float32)
        s2 = jnp.dot(out * out, ones, preferred_element_type=jnp.float32)
        mu = s1 * (1.0 / HID)
        var = s2 * (1.0 / HID) - mu * mu
        y = (out - mu) * jax.lax.rsqrt(var + 1e-5) * ln_g[l][None, :] + ln_b[l][None, :]
        x = y * jax.nn.sigmoid(y)

    pt = x.reshape(N * bb, T, HID).sum(axis=1)
    pooled = pt.reshape(N, bb, HID).sum(axis=0) * (1.0 / ROWS_PER_B)
    out_ref[...] = pooled


@jax.jit
def kernel(joint_0, joint_1, joint_2, joint_3, joint_4, joint_5, joint_6,
           eef, object, enc_w, enc_b, Wl, bl, Wr, br, att, We, Wres, bo,
           ln_g, ln_b):
    bb = 8
    k = B // bb
    feats = [joint_0, joint_1, joint_2, joint_3, joint_4, joint_5, joint_6,
             eef, object]
    feat_spec = pl.BlockSpec((bb, T, FEAT), lambda g: (g, 0, 0))

    def full(shape):
        nd = len(shape)
        return pl.BlockSpec(shape, lambda g: (0,) * nd)

    in_specs = [feat_spec] * N + [
        full(enc_w.shape), full(enc_b.shape), full(Wl.shape), full(bl.shape),
        full(Wr.shape), full(br.shape), full(att.shape), full(We.shape),
        full(Wres.shape), full(bo.shape), full(ln_g.shape), full(ln_b.shape),
        full((N, 4)),
    ]
    static = jnp.asarray(_STATIC)
    out = pl.pallas_call(
        functools.partial(_gat_kernel, bb=bb),
        grid=(k,),
        compiler_params=pltpu.CompilerParams(
            dimension_semantics=("parallel",)),
        in_specs=in_specs,
        out_specs=pl.BlockSpec((bb, HID), lambda g: (g, 0)),
        out_shape=jax.ShapeDtypeStruct((B, HID), jnp.float32),
    )(*feats, enc_w, enc_b, Wl, bl, Wr, br, att, We, Wres, bo, ln_g, ln_b,
      static)
    return out


# speedup vs baseline: 1.3118x; 1.0299x over previous
import functools

import jax
import jax.numpy as jnp
import numpy as np
from jax.experimental import pallas as pl
from jax.experimental.pallas import tpu as pltpu

B, T, N = 128, 64, 9
FEAT = 16
L, H, C = 3, 4, 32
HID = H * C
ENC = 128
ROWS_PER_B = T * N

_node_types = np.array([0, 0, 0, 0, 0, 0, 0, 1, 2])
_onehot = np.eye(3, dtype=np.float32)[_node_types]
_dist = np.abs(np.arange(N) - 7).astype(np.float32)
_STATIC = np.concatenate([_onehot, _dist[:, None]], axis=1)

_NEG = -1e30


def _gat_kernel(j0, j1, j2, j3, j4, j5, j6, j7, j8,
                enc_w, enc_b, Wl, bl, Wr, br, att, We, Wres, bo, ln_g, ln_b,
                static, out_ref, *, bb):
    bt = bb * T
    R = N * bt
    feats = [j0, j1, j2, j3, j4, j5, j6, j7, j8]

    blk = (jax.lax.broadcasted_iota(jnp.int32, (HID, HID), 0) // C ==
           jax.lax.broadcasted_iota(jnp.int32, (HID, HID), 1) // C
           ).astype(jnp.float32)
    ones = jnp.full((HID, HID), 1.0 / HID, jnp.float32)

    xs = []
    ps = []
    for i in range(N):
        f = feats[i][...].reshape(bt, FEAT)
        const = static[i:i + 1, :] @ enc_w[i, FEAT:, :] + enc_b[i][None, :]
        xs.append(jnp.dot(f, enc_w[i, :FEAT, :],
                          preferred_element_type=jnp.float32) + const)
        ps.append(f[:, :3])
    x = jnp.concatenate(xs, axis=0)
    pos = jnp.concatenate(ps, axis=0)

    def sh_l(a, w):
        return jnp.concatenate([jnp.zeros((bt, w), jnp.float32), a[:-bt]], 0)

    def sh_r(a, w):
        return jnp.concatenate([a[bt:], jnp.zeros((bt, w), jnp.float32)], 0)

    def sh_t(a, w):
        return jnp.concatenate([jnp.zeros((1, w), jnp.float32), a[:-1]], 0)

    r0 = jax.lax.broadcasted_iota(jnp.int32, (R, HID), 0)
    am_l = jnp.where(r0 >= bt, 0.0, _NEG)
    am_r = jnp.where(r0 < R - bt, 0.0, _NEG)
    am_t = jnp.where(r0 % T != 0, 0.0, _NEG)

    r1 = jax.lax.broadcasted_iota(jnp.int32, (R, 1), 0)
    m_l3 = (r1 >= bt).astype(jnp.float32)
    m_r3 = (r1 < R - bt).astype(jnp.float32)
    m_t3 = (r1 % T != 0).astype(jnp.float32)
    deg = m_l3 + m_r3 + m_t3

    pos_l = sh_l(pos, 3)
    pos_r = sh_r(pos, 3)
    pos_t = sh_t(pos, 3)
    dp_smean = (m_l3 * pos_l + m_r3 * pos_r + m_t3 * pos_t) / deg

    eye = (jax.lax.broadcasted_iota(jnp.int32, (HID, HID), 0) ==
           jax.lax.broadcasted_iota(jnp.int32, (HID, HID), 1)
           ).astype(jnp.float32)

    for l in range(L):
        wcat = jnp.concatenate([Wl[l], Wr[l], Wres[l]], axis=1)
        bcat = jnp.concatenate([bl[l], br[l], bo[l]], axis=0)[None, :]
        xw = jnp.dot(x, wcat, preferred_element_type=jnp.float32) + bcat
        xl = xw[:, :HID]
        xr = xw[:, HID:2 * HID]
        res = xw[:, 2 * HID:]

        g = jnp.dot(pos, We[l], preferred_element_type=jnp.float32)
        eesm = jnp.dot(dp_smean, We[l], preferred_element_type=jnp.float32)

        P = xr + g
        D = xl - g
        zpad = jnp.zeros((bt, HID), jnp.float32)
        xlz = jnp.concatenate([zpad, xl, zpad], 0)
        dz = jnp.concatenate([zpad, D, zpad], 0)
        xl_l = xlz[:R]
        xl_r = xlz[2 * bt:]
        xl_t = xlz[bt - 1:bt - 1 + R]
        d_l = dz[:R]
        d_r = dz[2 * bt:]
        d_t = dz[bt - 1:bt - 1 + R]

        attf = att[l].reshape(1, HID) * np.float32(1.4426950408889634)
        attblk = jnp.dot(eye * attf, blk, preferred_element_type=jnp.float32)

        def logit(m):
            m = jnp.maximum(m, 0.2 * m)
            return jnp.dot(m, attblk, preferred_element_type=jnp.float32)

        e_l = jnp.exp2(logit(P + d_l) + am_l)
        e_r = jnp.exp2(logit(P + d_r) + am_r)
        e_t = jnp.exp2(logit(P + d_t) + am_t)
        e_s = jnp.exp2(logit(P + xl - eesm))
        inv = 1.0 / (e_l + e_r + e_t + e_s + 1e-16)

        out = (e_l * xl_l + e_r * xl_r + e_t * xl_t +
               e_s * xl) * inv + res

        mu = jnp.dot(out, ones, preferred_element_type=jnp.float32)
        msq = jnp.dot(out * out, ones, preferred_element_type=jnp.float32)
        var = msq - mu * mu
        y = (out - mu) * jax.lax.rsqrt(var + 1e-5) * ln_g[l][None, :] + ln_b[l][None, :]
        x = y * jax.nn.sigmoid(y)

    pt = x.reshape(N * bb, T, HID).sum(axis=1)
    pooled = pt.reshape(N, bb, HID).sum(axis=0) * (1.0 / ROWS_PER_B)
    out_ref[...] = pooled


@jax.jit
def kernel(joint_0, joint_1, joint_2, joint_3, joint_4, joint_5, joint_6,
           eef, object, enc_w, enc_b, Wl, bl, Wr, br, att, We, Wres, bo,
           ln_g, ln_b):
    bb = 8
    k = B // bb
    feats = [joint_0, joint_1, joint_2, joint_3, joint_4, joint_5, joint_6,
             eef, object]
    feat_spec = pl.BlockSpec((bb, T, FEAT), lambda g: (g, 0, 0))

    def full(shape):
        nd = len(shape)
        return pl.BlockSpec(shape, lambda g: (0,) * nd)

    in_specs = [feat_spec] * N + [
        full(enc_w.shape), full(enc_b.shape), full(Wl.shape), full(bl.shape),
        full(Wr.shape), full(br.shape), full(att.shape), full(We.shape),
        full(Wres.shape), full(bo.shape), full(ln_g.shape), full(ln_b.shape),
        full((N, 4)),
    ]
    static = jnp.asarray(_STATIC)
    out = pl.pallas_call(
        functools.partial(_gat_kernel, bb=bb),
        grid=(k,),
        compiler_params=pltpu.CompilerParams(
            dimension_semantics=("parallel",)),
        in_specs=in_specs,
        out_specs=pl.BlockSpec((bb, HID), lambda g: (g, 0)),
        out_shape=jax.ShapeDtypeStruct((B, HID), jnp.float32),
    )(*feats, enc_w, enc_b, Wl, bl, Wr, br, att, We, Wres, bo, ln_g, ln_b,
      static)
    return out
